# baseline (speedup 1.0000x reference)
import functools

import jax
import jax.numpy as jnp
from jax import lax
from jax.experimental import pallas as pl
from jax.experimental.pallas import tpu as pltpu
from jax.experimental.pallas import tpu_sc as plsc

NU = 50000
NI = 50000
NN = NU + NI
EE = 1600000
DH = 16
BB = 16384
KK = 3
NC = 2
NS = 16
LL = 16

CH = 1024
CR = CH // 128
NQSUB = 98
EP = NS * NQSUB * CH
EPX = EP + NS * CH
ER = EPX // 128

SPAN = 6256
SPAN_LAST = NN - 15 * SPAN
ACC_ROWS = NS * SPAN

PP = BB // (NC * NS)
PR = PP // 128

_mesh = plsc.VectorSubcoreMesh(core_axis_name="c", subcore_axis_name="s")

_GDN = lax.GatherDimensionNumbers(
    offset_dims=(), collapsed_slice_dims=(0,), start_index_map=(0,))


def _bcast_lane(v, k):
    idx = jnp.full((LL, 1), k, jnp.int32)
    return lax.gather(v, idx, _GDN, (1,),
                      mode=lax.GatherScatterMode.PROMISE_IN_BOUNDS)


@functools.partial(
    pl.kernel,
    out_type=jax.ShapeDtypeStruct((KK * 2 * NN, DH), jnp.float32),
    mesh=_mesh,
    compiler_params=pltpu.CompilerParams(use_tc_tiling_on_sc=False),
    scratch_types=[
        pltpu.VMEM((CR, 128), jnp.int32),
        pltpu.VMEM((CR, 128), jnp.int32),
        pltpu.VMEM((CR, 128), jnp.int32),
        pltpu.VMEM((CR, 128), jnp.int32),
        pltpu.VMEM((CH,), jnp.float32),
        pltpu.VMEM((CH,), jnp.float32),
        pltpu.VMEM((CH, DH), jnp.float32),
        pltpu.VMEM_SHARED((ACC_ROWS, DH), jnp.float32),
        pltpu.SemaphoreType.DMA,
        pltpu.SemaphoreType.DMA,
        pltpu.SemaphoreType.DMA,
        pltpu.SemaphoreType.REGULAR,
    ],
)
def _hops(h2, srcr, dstr, vals, zeros, touts, srcbuf0, srcbuf1, dstbuf0,
          dstbuf1, valbuf0, valbuf1, rows, acc, sem_in, sem_g, sem_s, csem):
    c = lax.axis_index("c")
    s = lax.axis_index("s")
    base = s * SPAN
    bufs = ((srcbuf0, dstbuf0, valbuf0), (srcbuf1, dstbuf1, valbuf1))

    def _run_hop(gref, sblock, wslot, last):
        pltpu.sync_copy(zeros, acc.at[pl.ds(base, SPAN)])
        plsc.subcore_barrier()
        srbase = (2 * sblock + c) * ER

        def _fire_in(t, phase):
            q = s + NS * (2 * t + phase)
            ro = CR * q
            sb, db, vb = bufs[phase % 2]
            pltpu.async_copy(srcr.at[pl.ds(srbase + ro, CR)], sb, sem_in)
            pltpu.async_copy(dstr.at[pl.ds(ro, CR)], db, sem_in)
            pltpu.async_copy(vals.at[pl.ds(CH * q, CH)], vb, sem_in)

        def _drain_in(phase):
            sb, db, vb = bufs[phase % 2]
            pltpu.make_async_copy(srcr.at[pl.ds(0, CR)], sb, sem_in).wait()
            pltpu.make_async_copy(dstr.at[pl.ds(0, CR)], db, sem_in).wait()
            pltpu.make_async_copy(vals.at[pl.ds(0, CH)], vb, sem_in).wait()

        _fire_in(0, 0)

        @pl.loop(0, NQSUB // 2)
        def _pair(t):
            for phase in range(2):
                sb, db, vb = bufs[phase]
                if phase == 0:
                    _fire_in(t, 1)
                else:
                    @pl.when(t < NQSUB // 2 - 1)
                    def _prefetch_next_pair():
                        _fire_in(t + 1, 0)
                _drain_in(phase)
                gs = [None] * CR

                def _fire_g(j):
                    gs[j] = pltpu.async_copy(
                        gref.at[sb.at[j]],
                        rows.at[pl.ds(128 * j, 128)], sem_g)

                _fire_g(0)
                sc = [None] * CR
                for j in range(CR):
                    if j + 1 < CR:
                        _fire_g(j + 1)
                    gs[j].wait()

                    @pl.loop(0, 128 // LL)
                    def _scale(g, j=j):
                        gp = (128 // LL) * j + g
                        v16 = vb[pl.ds(LL * gp, LL)]
                        for k in range(LL):
                            e = LL * gp + k
                            rows[e, :] = rows[e, :] * _bcast_lane(v16, k)

                    sc[j] = pltpu.async_copy(rows.at[pl.ds(128 * j, 128)],
                                             acc.at[db.at[j]], sem_s, add=True)
                for j in range(CR):
                    sc[j].wait()

        plsc.subcore_barrier()
        woff = wslot * 2 * NN + c * NN + base

        @pl.when(s < NS - 1)
        def _store_main():
            pltpu.sync_copy(acc.at[pl.ds(base, SPAN)],
                            touts.at[pl.ds(woff, SPAN)])

        @pl.when(s == NS - 1)
        def _store_last():
            pltpu.sync_copy(acc.at[pl.ds(base, SPAN_LAST)],
                            touts.at[pl.ds(woff, SPAN_LAST)])

        if not last:
            plsc.subcore_barrier()

            @pl.when(s == 0)
            def _cross_core():
                pltpu.core_barrier(csem, core_axis_name="c")

            plsc.subcore_barrier()

    _run_hop(h2, 0, 0, False)
    _run_hop(touts, 0, 1, False)
    _run_hop(touts, 1, 2, True)


@functools.partial(
    pl.kernel,
    out_type=jax.ShapeDtypeStruct((BB,), jnp.float32),
    mesh=_mesh,
    compiler_params=pltpu.CompilerParams(use_tc_tiling_on_sc=False,
                                         needs_layout_passes=False),
    scratch_types=[
        pltpu.VMEM((PP,), jnp.int32),
        pltpu.VMEM((PP,), jnp.int32),
        pltpu.VMEM((PP,), jnp.int32),
        pltpu.VMEM((PP,), jnp.int32),
        pltpu.VMEM((PP,), jnp.int32),
        pltpu.VMEM((PP, DH), jnp.float32),
        pltpu.VMEM((PP, DH), jnp.float32),
        pltpu.VMEM((PP, DH), jnp.float32),
        pltpu.VMEM((PP, DH), jnp.float32),
        pltpu.VMEM((PP, DH), jnp.float32),
        pltpu.VMEM((PP,), jnp.float32),
        pltpu.SemaphoreType.DMA,
    ],
)
def _final(h2, touts, users, items, out,
           iuA, iuB, ivA, ivB, ishift, usA, usB, vsA, vsB, gbuf, sbuf, sem):
    c = lax.axis_index("c")
    s = lax.axis_index("s")
    wid = c * NS + s
    eo = PP * wid
    cp1 = pltpu.async_copy(users.at[pl.ds(eo, PP)], iuA, sem)
    cp2 = pltpu.async_copy(items.at[pl.ds(eo, PP)], ivA, sem)
    cp1.wait()
    cp2.wait()
    offN = jnp.full((LL,), NN, jnp.int32)
    offI = jnp.full((LL,), NU, jnp.int32)
    for k in range(PP // LL):
        sl = pl.ds(LL * k, LL)
        u = iuA[sl]
        iuB[sl] = u + offN
        v = ivA[sl] + offI
        ivA[sl] = v
        ivB[sl] = v + offN

    def _gather_sum(sumbuf, idxbuf):
        cps = [pltpu.async_copy(h2.at[idxbuf.at[pl.ds(128 * j, 128)]],
                                sumbuf.at[pl.ds(128 * j, 128)], sem)
               for j in range(PR)]
        for cp in cps:
            cp.wait()
        for tk in range(KK):
            toff = jnp.full((LL,), tk * 2 * NN, jnp.int32)
            for k in range(PP // LL):
                sl = pl.ds(LL * k, LL)
                ishift[sl] = idxbuf[sl] + toff
            cps = [pltpu.async_copy(touts.at[ishift.at[pl.ds(128 * j, 128)]],
                                    gbuf.at[pl.ds(128 * j, 128)], sem)
                   for j in range(PR)]
            for cp in cps:
                cp.wait()

            @pl.loop(0, PP)
            def _accum(p):
                sumbuf[p, :] = sumbuf[p, :] + gbuf[p, :]

    _gather_sum(usA, iuA)
    _gather_sum(usB, iuB)
    _gather_sum(vsA, ivA)
    _gather_sum(vsB, ivB)

    lane = jnp.arange(LL, dtype=jnp.int32)
    scale = jnp.float32(1.0 / 16.0)

    @pl.loop(0, PP // LL)
    def _dot(gp):
        out16 = jnp.zeros((LL,), jnp.float32)
        for k in range(LL):
            p = LL * gp + k
            pa = usA[p, :] * vsA[p, :] + usB[p, :] * vsB[p, :]
            sc = jnp.sum(pa) * scale
            out16 = jnp.where(lane == k, sc, out16)
        sbuf[pl.ds(LL * gp, LL)] = out16

    pltpu.sync_copy(sbuf, out.at[pl.ds(PP * wid, PP)])


def kernel(users, items, edge_index, edge_vals, user_emb, item_emb):
    h2 = jnp.concatenate([user_emb[:, :DH], item_emb[:, :DH],
                          user_emb[:, DH:], item_emb[:, DH:]], axis=0)
    pad = EPX - EE
    zpad_i = jnp.zeros((pad,), jnp.int32)
    src0 = jnp.concatenate([edge_index[0], zpad_i]).reshape(ER, 128)
    srcr = jnp.concatenate([src0, src0 + NN, src0 + 2 * NN, src0 + 3 * NN],
                           axis=0)
    dstr = jnp.concatenate([edge_index[1], zpad_i]).reshape(ER, 128)
    valsp = jnp.concatenate([edge_vals, jnp.zeros((pad,), jnp.float32)])
    zeros = jnp.zeros((SPAN, DH), jnp.float32)
    touts = _hops(h2, srcr, dstr, valsp, zeros)
    return _final(h2, touts, users, items)


# speedup vs baseline: 1.4698x; 1.4698x over previous
import functools

import jax
import jax.numpy as jnp
from jax import lax
from jax.experimental import pallas as pl
from jax.experimental.pallas import tpu as pltpu
from jax.experimental.pallas import tpu_sc as plsc

NU = 50000
NI = 50000
NN = NU + NI
EE = 1600000
DH = 16
BB = 16384
NC = 2
NS = 16
LL = 16

CH = 1024
CR = CH // 128
NQSUB = 98
EP = NS * NQSUB * CH
EPX = EP + NS * CH
ER = EPX // 128

SPAN = 6256
SPAN_LAST = NN - 15 * SPAN
ACC_ROWS = NS * SPAN

PP = BB // (NC * NS)
PR = PP // 128

_mesh = plsc.VectorSubcoreMesh(core_axis_name="c", subcore_axis_name="s")

_GDN = lax.GatherDimensionNumbers(
    offset_dims=(), collapsed_slice_dims=(0,), start_index_map=(0,))


def _bcast_lane(v, k):
    idx = jnp.full((LL, 1), k, jnp.int32)
    return lax.gather(v, idx, _GDN, (1,),
                      mode=lax.GatherScatterMode.PROMISE_IN_BOUNDS)


@functools.partial(
    pl.kernel,
    out_type=jax.ShapeDtypeStruct((2 * NN, DH), jnp.float32),
    mesh=_mesh,
    compiler_params=pltpu.CompilerParams(use_tc_tiling_on_sc=False),
    scratch_types=[
        pltpu.VMEM((CR, 128), jnp.int32),
        pltpu.VMEM((CR, 128), jnp.int32),
        pltpu.VMEM((CR, 128), jnp.int32),
        pltpu.VMEM((CR, 128), jnp.int32),
        pltpu.VMEM((CH,), jnp.float32),
        pltpu.VMEM((CH,), jnp.float32),
        pltpu.VMEM((CH, DH), jnp.float32),
        pltpu.VMEM_SHARED((ACC_ROWS, DH), jnp.float32),
        pltpu.SemaphoreType.DMA,
        pltpu.SemaphoreType.DMA,
        pltpu.SemaphoreType.DMA,
    ],
)
def _hop(h2, srcr, dstr, vals, zeros, out, srcbuf0, srcbuf1, dstbuf0, dstbuf1,
         valbuf0, valbuf1, rows, acc, sem_in, sem_g, sem_s):
    c = lax.axis_index("c")
    s = lax.axis_index("s")

    base = s * SPAN
    pltpu.sync_copy(zeros, acc.at[pl.ds(base, SPAN)])
    plsc.subcore_barrier()

    bufs = ((srcbuf0, dstbuf0, valbuf0), (srcbuf1, dstbuf1, valbuf1))
    srbase = c * ER

    def _fire_in(t, phase):
        q = s + NS * (2 * t + phase)
        ro = CR * q
        eo = CH * q
        sb, db, vb = bufs[phase % 2]
        pltpu.async_copy(srcr.at[pl.ds(srbase + ro, CR)], sb, sem_in)
        pltpu.async_copy(dstr.at[pl.ds(ro, CR)], db, sem_in)
        pltpu.async_copy(vals.at[pl.ds(eo, CH)], vb, sem_in)

    def _drain_in(phase):
        sb, db, vb = bufs[phase % 2]
        pltpu.make_async_copy(srcr.at[pl.ds(0, CR)], sb, sem_in).wait()
        pltpu.make_async_copy(dstr.at[pl.ds(0, CR)], db, sem_in).wait()
        pltpu.make_async_copy(vals.at[pl.ds(0, CH)], vb, sem_in).wait()

    _fire_in(0, 0)

    @pl.loop(0, NQSUB // 2)
    def _pair(t):
        for phase in range(2):
            sb, db, vb = bufs[phase]
            if phase == 0:
                _fire_in(t, 1)
            else:
                @pl.when(t < NQSUB // 2 - 1)
                def _prefetch_next_pair():
                    _fire_in(t + 1, 0)
            _drain_in(phase)
            gs = [pltpu.async_copy(h2.at[sb.at[j]],
                                   rows.at[pl.ds(128 * j, 128)], sem_g)
                  for j in range(CR)]
            sc = [None] * CR
            for j in range(CR):
                gs[j].wait()

                @pl.loop(0, 128 // LL)
                def _scale(g, j=j):
                    gp = (128 // LL) * j + g
                    v16 = vb[pl.ds(LL * gp, LL)]
                    for k in range(LL):
                        e = LL * gp + k
                        rows[e, :] = rows[e, :] * _bcast_lane(v16, k)

                sc[j] = pltpu.async_copy(rows.at[pl.ds(128 * j, 128)],
                                         acc.at[db.at[j]], sem_s, add=True)
            for j in range(CR):
                sc[j].wait()

    plsc.subcore_barrier()

    @pl.when(s < NS - 1)
    def _store_main():
        pltpu.sync_copy(acc.at[pl.ds(base, SPAN)],
                        out.at[pl.ds(c * NN + base, SPAN)])

    @pl.when(s == NS - 1)
    def _store_last():
        pltpu.sync_copy(acc.at[pl.ds(base, SPAN_LAST)],
                        out.at[pl.ds(c * NN + base, SPAN_LAST)])


@functools.partial(
    pl.kernel,
    out_type=jax.ShapeDtypeStruct((BB,), jnp.float32),
    mesh=_mesh,
    compiler_params=pltpu.CompilerParams(use_tc_tiling_on_sc=False, needs_layout_passes=False),
    scratch_types=[
        pltpu.VMEM((PP,), jnp.int32),
        pltpu.VMEM((PP,), jnp.int32),
        pltpu.VMEM((PP,), jnp.int32),
        pltpu.VMEM((PP,), jnp.int32),
        pltpu.VMEM((PP, DH), jnp.float32),
        pltpu.VMEM((PP, DH), jnp.float32),
        pltpu.VMEM((PP, DH), jnp.float32),
        pltpu.VMEM((PP, DH), jnp.float32),
        pltpu.VMEM((PP, DH), jnp.float32),
        pltpu.VMEM((PP,), jnp.float32),
        pltpu.SemaphoreType.DMA,
    ],
)
def _final(t0, t1, t2, t3, users, items, out,
           iuA, iuB, ivA, ivB, usA, usB, vsA, vsB, gbuf, sbuf, sem):
    c = lax.axis_index("c")
    s = lax.axis_index("s")
    wid = c * NS + s
    eo = PP * wid
    cp1 = pltpu.async_copy(users.at[pl.ds(eo, PP)], iuA, sem)
    cp2 = pltpu.async_copy(items.at[pl.ds(eo, PP)], ivA, sem)
    cp1.wait()
    cp2.wait()
    offN = jnp.full((LL,), NN, jnp.int32)
    offI = jnp.full((LL,), NU, jnp.int32)
    for k in range(PP // LL):
        sl = pl.ds(LL * k, LL)
        u = iuA[sl]
        iuB[sl] = u + offN
        v = ivA[sl] + offI
        ivA[sl] = v
        ivB[sl] = v + offN

    tabs = (t0, t1, t2, t3)
    for sumbuf, idxbuf in ((usA, iuA), (usB, iuB), (vsA, ivA), (vsB, ivB)):
        cps = [pltpu.async_copy(tabs[0].at[idxbuf.at[pl.ds(128 * j, 128)]],
                                sumbuf.at[pl.ds(128 * j, 128)], sem)
               for j in range(PR)]
        for cp in cps:
            cp.wait()
        for tk in tabs[1:]:
            cps = [pltpu.async_copy(tk.at[idxbuf.at[pl.ds(128 * j, 128)]],
                                    gbuf.at[pl.ds(128 * j, 128)], sem)
                   for j in range(PR)]
            for cp in cps:
                cp.wait()

            @pl.loop(0, PP)
            def _accum(p):
                sumbuf[p, :] = sumbuf[p, :] + gbuf[p, :]

    lane = jnp.arange(LL, dtype=jnp.int32)
    scale = jnp.float32(1.0 / 16.0)

    @pl.loop(0, PP // LL)
    def _dot(gp):
        out16 = jnp.zeros((LL,), jnp.float32)
        for k in range(LL):
            p = LL * gp + k
            pa = usA[p, :] * vsA[p, :] + usB[p, :] * vsB[p, :]
            sc = jnp.sum(pa) * scale
            out16 = jnp.where(lane == k, sc, out16)
        sbuf[pl.ds(LL * gp, LL)] = out16

    pltpu.sync_copy(sbuf, out.at[pl.ds(PP * wid, PP)])


def kernel(users, items, edge_index, edge_vals, user_emb, item_emb):
    all_ego = jnp.concatenate([user_emb, item_emb], axis=0)
    h2 = jnp.concatenate([all_ego[:, :DH], all_ego[:, DH:]], axis=0)
    pad = EPX - EE
    zpad_i = jnp.zeros((pad,), jnp.int32)
    src0 = jnp.concatenate([edge_index[0], zpad_i]).reshape(ER, 128)
    srcr = jnp.concatenate([src0, src0 + NN], axis=0)
    dstr = jnp.concatenate([edge_index[1], zpad_i]).reshape(ER, 128)
    valsp = jnp.concatenate([edge_vals, jnp.zeros((pad,), jnp.float32)])
    zeros = jnp.zeros((SPAN, DH), jnp.float32)
    t0 = h2
    t1 = _hop(t0, srcr, dstr, valsp, zeros)
    t2 = _hop(t1, srcr, dstr, valsp, zeros)
    t3 = _hop(t2, srcr, dstr, valsp, zeros)
    return _final(t0, t1, t2, t3, users, items)
